# baseline (speedup 1.0000x reference)
import functools

import jax
import jax.numpy as jnp
from jax import lax
from jax.experimental import pallas as pl
from jax.experimental.pallas import tpu as pltpu
from jax.experimental.pallas import tpu_sc as plsc

NC, NS, L = 2, 16, 16
NW = NC * NS
B = 2000
NPAD = 102400

_MESH = plsc.VectorSubcoreMesh(
    core_axis_name="c", subcore_axis_name="s", num_cores=NC, num_subcores=NS)


def _make_deg_kernel(E):
    nb = E // NW // B
    SL = NPAD // NS
    assert nb % 4 == 0 and nb >= 4 and B % L == 0

    scratch = ([pltpu.VMEM((B,), jnp.int32) for _ in range(4)]
               + [pltpu.VMEM((B,), jnp.float32),
                  pltpu.VMEM_SHARED((NPAD,), jnp.float32)]
               + [pltpu.SemaphoreType.DMA for _ in range(6)])

    @functools.partial(
        pl.kernel,
        out_type=jax.ShapeDtypeStruct((NC * NPAD,), jnp.float32),
        mesh=_MESH,
        scratch_types=scratch,
    )
    def deg_k(dst1, zeros, out, *rest):
        didx = rest[0:4]
        ones_v, acc = rest[4], rest[5]
        ls = rest[6:10]
        ss = rest[10:12]
        c = lax.axis_index("c")
        s = lax.axis_index("s")
        base = (c * NS + s) * (nb * B)

        def load(j, r4):
            pltpu.async_copy(dst1.at[pl.ds(base + j * B, B)], didx[r4], ls[r4])

        for r in range(2):
            load(r, r)
        pltpu.sync_copy(zeros.at[pl.ds(s * SL, SL)], acc.at[pl.ds(s * SL, SL)])
        for i in range(B // L):
            ones_v[pl.ds(i * L, L)] = jnp.ones((L,), jnp.float32)
        plsc.subcore_barrier()

        def group(g, cc):
            for rr in range(4):
                j = g * 4 + rr
                r4 = rr
                jw = jnp.where(j + 2 < nb, j + 2, j + 2 - nb)
                load(jw, (rr + 2) % 4)
                pltpu.make_async_copy(
                    dst1.at[pl.ds(base + j * B, B)], didx[r4], ls[r4]).wait()
                pltpu.sync_copy(ones_v, acc.at[didx[r4]], add=True)
            return cc

        lax.fori_loop(0, nb // 4, group, 0)
        for r in range(2):
            pltpu.make_async_copy(
                dst1.at[pl.ds(base, B)], didx[r], ls[r]).wait()
        plsc.subcore_barrier()
        pltpu.sync_copy(acc.at[pl.ds(s * SL, SL)],
                        out.at[pl.ds(c * NPAD + s * SL, SL)])

    return deg_k


def _make_agg_kernel(E, T):
    nb = E // NW // B
    SL = NPAD // NS
    assert nb % 4 == 0 and nb >= 4

    scratch = ([pltpu.VMEM((B,), jnp.int32) for _ in range(8)]
               + [pltpu.VMEM((B,), jnp.float32) for _ in range(2 * T)]
               + [pltpu.VMEM_SHARED((NPAD,), jnp.float32) for _ in range(2 * T)]
               + [pltpu.SemaphoreType.DMA for _ in range(8)])

    @functools.partial(
        pl.kernel,
        out_type=tuple(jax.ShapeDtypeStruct((NC * NPAD,), jnp.float32)
                       for _ in range(T)),
        mesh=_MESH,
        scratch_types=scratch,
    )
    def agg_k(src1, dst1, *rest):
        tabs = rest[:T]
        zeros = rest[T]
        outs = rest[T + 1:T + 1 + T]
        sc = list(rest[T + 1 + T:])
        sidx = sc[0:4]
        didx = sc[4:8]
        rows = [sc[8 + 2 * t:8 + 2 * t + 2] for t in range(T)]
        accs = sc[8 + 2 * T:8 + 3 * T]
        tabs_sh = sc[8 + 3 * T:8 + 4 * T]
        ls = sc[8 + 4 * T:12 + 4 * T]
        sg = sc[12 + 4 * T:14 + 4 * T]
        ss = sc[14 + 4 * T:16 + 4 * T]
        c = lax.axis_index("c")
        s = lax.axis_index("s")
        base = (c * NS + s) * (nb * B)

        def load(j, r4):
            pltpu.async_copy(src1.at[pl.ds(base + j * B, B)], sidx[r4], ls[r4])
            pltpu.async_copy(dst1.at[pl.ds(base + j * B, B)], didx[r4], ls[r4])

        def wait_load(j, r4):
            pltpu.make_async_copy(
                src1.at[pl.ds(base + j * B, B)], sidx[r4], ls[r4]).wait()
            pltpu.make_async_copy(
                dst1.at[pl.ds(base + j * B, B)], didx[r4], ls[r4]).wait()

        def wait_scatter(r2, r4):
            for t in range(T):
                pltpu.make_async_copy(
                    rows[t][r2], accs[t].at[didx[r4]], ss[r2]).wait()

        for r in range(2):
            load(r, r)
        for t in range(T):
            pltpu.sync_copy(zeros.at[pl.ds(s * SL, SL)],
                            accs[t].at[pl.ds(s * SL, SL)])
            pltpu.sync_copy(tabs[t].at[pl.ds(s * SL, SL)],
                            tabs_sh[t].at[pl.ds(s * SL, SL)])
        plsc.subcore_barrier()

        def group(g, cc):
            for rr in range(4):
                j = g * 4 + rr
                r2, r4 = rr % 2, rr
                jw = jnp.where(j + 2 < nb, j + 2, j + 2 - nb)
                load(jw, (rr + 2) % 4)
                wait_load(j, r4)
                for t in range(T):
                    pltpu.sync_copy(tabs_sh[t].at[sidx[r4]], rows[t][r2])
                for t in range(T):
                    pltpu.sync_copy(rows[t][r2], accs[t].at[didx[r4]],
                                    add=True)
            return cc

        lax.fori_loop(0, nb // 4, group, 0)
        for r in range(2):
            wait_load(0, r)
        plsc.subcore_barrier()
        for t in range(T):
            pltpu.sync_copy(accs[t].at[pl.ds(s * SL, SL)],
                            outs[t].at[pl.ds(c * NPAD + s * SL, SL)])

    return agg_k


_LB = 2048


def _tc_norm(deg_p, x_t):
    G = NPAD // _LB

    def body(dref, xref, dinvref, uref):
        deg = dref[0:1, :] + dref[1:2, :] + 1.0
        dinv = lax.rsqrt(deg)
        dinvref[...] = dinv
        uref[...] = xref[...] * dinv

    return pl.pallas_call(
        body,
        grid=(G,),
        in_specs=[pl.BlockSpec((NC, _LB), lambda i: (0, i)),
                  pl.BlockSpec((2, _LB), lambda i: (0, i))],
        out_specs=[pl.BlockSpec((1, _LB), lambda i: (0, i)),
                   pl.BlockSpec((2, _LB), lambda i: (0, i))],
        out_shape=[jax.ShapeDtypeStruct((1, NPAD), jnp.float32),
                   jax.ShapeDtypeStruct((2, NPAD), jnp.float32)],
    )(deg_p, x_t)


def _tc_layer1(a0p, a1p, x_t, dinv, wpack):
    G = NPAD // _LB

    def body(a0r, a1r, xr, dr, wr, vr, gr):
        dinv = dr[...]
        d2 = dinv * dinv
        a0 = dinv * (a0r[0:1, :] + a0r[1:2, :]) + d2 * xr[0:1, :]
        a1 = dinv * (a1r[0:1, :] + a1r[1:2, :]) + d2 * xr[1:2, :]
        w = wr[...]
        h = jnp.maximum(w[:, 0:1] * a0 + w[:, 1:2] * a1 + w[:, 2:3], 0.0)
        g = jnp.sum(h * w[:, 3:4], axis=0, keepdims=True)
        gr[...] = g
        vr[...] = dinv * g

    return pl.pallas_call(
        body,
        grid=(G,),
        in_specs=[pl.BlockSpec((NC, _LB), lambda i: (0, i)),
                  pl.BlockSpec((NC, _LB), lambda i: (0, i)),
                  pl.BlockSpec((2, _LB), lambda i: (0, i)),
                  pl.BlockSpec((1, _LB), lambda i: (0, i)),
                  pl.BlockSpec((16, 4), lambda i: (0, 0))],
        out_specs=[pl.BlockSpec((1, _LB), lambda i: (0, i)),
                   pl.BlockSpec((1, _LB), lambda i: (0, i))],
        out_shape=[jax.ShapeDtypeStruct((1, NPAD), jnp.float32),
                   jax.ShapeDtypeStruct((1, NPAD), jnp.float32)],
    )(a0p, a1p, x_t, dinv, wpack)


def _tc_layer2(a2p, dinv, g, b2):
    G = NPAD // _LB

    def body(ar, dr, gr, br, outr):
        dinv = dr[...]
        outr[...] = dinv * (ar[0:1, :] + ar[1:2, :]) + dinv * dinv * gr[...] + br[...]

    return pl.pallas_call(
        body,
        grid=(G,),
        in_specs=[pl.BlockSpec((NC, _LB), lambda i: (0, i)),
                  pl.BlockSpec((1, _LB), lambda i: (0, i)),
                  pl.BlockSpec((1, _LB), lambda i: (0, i)),
                  pl.BlockSpec((1, 1), lambda i: (0, 0))],
        out_specs=pl.BlockSpec((1, _LB), lambda i: (0, i)),
        out_shape=jax.ShapeDtypeStruct((1, NPAD), jnp.float32),
    )(a2p, dinv, g, b2)


def kernel(x, edge_index, batch, W1, b1, W2, b2):
    N = x.shape[0]
    E = edge_index.shape[1]
    assert E % (NW * B) == 0 and N <= NPAD

    src1 = edge_index[0]
    dst1 = edge_index[1]
    zeros = jnp.zeros((NPAD,), jnp.float32)
    x_t = jnp.zeros((2, NPAD), jnp.float32).at[:, :N].set(x.T)

    deg_p = _make_deg_kernel(E)(dst1, zeros).reshape(NC, NPAD)
    dinv, u_t = _tc_norm(deg_p, x_t)

    a0p, a1p = _make_agg_kernel(E, 2)(
        src1, dst1, u_t[0].reshape(NPAD), u_t[1].reshape(NPAD), zeros)
    wpack = jnp.stack([W1[0], W1[1], b1, W2[:, 0]], axis=1)
    v, g = _tc_layer1(a0p.reshape(NC, NPAD), a1p.reshape(NC, NPAD),
                      x_t, dinv, wpack)

    (a2p,) = _make_agg_kernel(E, 1)(src1, dst1, v.reshape(NPAD), zeros)
    out = _tc_layer2(a2p.reshape(NC, NPAD), dinv, g, b2.reshape(1, 1))
    return out[0, :N].reshape(N, 1)


# speedup vs baseline: 1.5363x; 1.5363x over previous
import functools

import jax
import jax.numpy as jnp
from jax import lax
from jax.experimental import pallas as pl
from jax.experimental.pallas import tpu as pltpu
from jax.experimental.pallas import tpu_sc as plsc

NC, NS, L = 2, 16, 16
NW = NC * NS
B = 10000
NPAD = 102400

_MESH = plsc.VectorSubcoreMesh(
    core_axis_name="c", subcore_axis_name="s", num_cores=NC, num_subcores=NS)


def _make_deg_kernel(E):
    nb = E // NW // B
    SL = NPAD // NS
    assert nb % 2 == 0 and nb >= 2 and B % L == 0

    scratch = ([pltpu.VMEM((B,), jnp.int32) for _ in range(2)]
               + [pltpu.VMEM((B,), jnp.float32),
                  pltpu.VMEM_SHARED((NPAD,), jnp.float32)]
               + [pltpu.SemaphoreType.DMA for _ in range(2)])

    @functools.partial(
        pl.kernel,
        out_type=jax.ShapeDtypeStruct((NC * NPAD,), jnp.float32),
        mesh=_MESH,
        scratch_types=scratch,
    )
    def deg_k(dst1, zeros, out, *rest):
        didx = rest[0:2]
        ones_v, acc = rest[2], rest[3]
        ls = rest[4:6]
        c = lax.axis_index("c")
        s = lax.axis_index("s")
        base = (c * NS + s) * (nb * B)

        def load(j, r):
            pltpu.async_copy(dst1.at[pl.ds(base + j * B, B)], didx[r], ls[r])

        load(0, 0)
        pltpu.sync_copy(zeros.at[pl.ds(s * SL, SL)], acc.at[pl.ds(s * SL, SL)])
        for i in range(B // L):
            ones_v[pl.ds(i * L, L)] = jnp.ones((L,), jnp.float32)
        plsc.subcore_barrier()

        def group(g, cc):
            for rr in range(2):
                j = g * 2 + rr
                jw = jnp.where(j + 1 < nb, j + 1, 0)
                load(jw, 1 - rr)
                pltpu.make_async_copy(
                    dst1.at[pl.ds(base + j * B, B)], didx[rr], ls[rr]).wait()
                pltpu.sync_copy(ones_v, acc.at[didx[rr]], add=True)
            return cc

        lax.fori_loop(0, nb // 2, group, 0)
        pltpu.make_async_copy(dst1.at[pl.ds(base, B)], didx[0], ls[0]).wait()
        plsc.subcore_barrier()
        pltpu.sync_copy(acc.at[pl.ds(s * SL, SL)],
                        out.at[pl.ds(c * NPAD + s * SL, SL)])

    return deg_k


def _make_agg_kernel(E, T):
    nb = E // NW // B
    SL = NPAD // NS
    assert nb % 2 == 0 and nb >= 2 and B % L == 0

    scratch = ([pltpu.VMEM((B,), jnp.int32) for _ in range(4)]
               + [pltpu.VMEM((B,), jnp.float32) for _ in range(T)]
               + [pltpu.VMEM_SHARED((NPAD,), jnp.float32) for _ in range(2 * T)]
               + [pltpu.SemaphoreType.DMA for _ in range(2)])

    @functools.partial(
        pl.kernel,
        out_type=tuple(jax.ShapeDtypeStruct((NC * NPAD,), jnp.float32)
                       for _ in range(T)),
        mesh=_MESH,
        scratch_types=scratch,
    )
    def agg_k(src1, dst1, *rest):
        tabs = rest[:T]
        zeros = rest[T]
        outs = rest[T + 1:T + 1 + T]
        sc = list(rest[T + 1 + T:])
        sidx = sc[0:2]
        didx = sc[2:4]
        rows = sc[4:4 + T]
        accs = sc[4 + T:4 + 2 * T]
        tabs_sh = sc[4 + 2 * T:4 + 3 * T]
        ls = sc[4 + 3 * T:6 + 3 * T]
        c = lax.axis_index("c")
        s = lax.axis_index("s")
        base = (c * NS + s) * (nb * B)

        def load(j, r):
            pltpu.async_copy(src1.at[pl.ds(base + j * B, B)], sidx[r], ls[r])
            pltpu.async_copy(dst1.at[pl.ds(base + j * B, B)], didx[r], ls[r])

        def wait_load(j, r):
            pltpu.make_async_copy(
                src1.at[pl.ds(base + j * B, B)], sidx[r], ls[r]).wait()
            pltpu.make_async_copy(
                dst1.at[pl.ds(base + j * B, B)], didx[r], ls[r]).wait()

        load(0, 0)
        for t in range(T):
            pltpu.sync_copy(zeros.at[pl.ds(s * SL, SL)],
                            accs[t].at[pl.ds(s * SL, SL)])
            pltpu.sync_copy(tabs[t].at[pl.ds(s * SL, SL)],
                            tabs_sh[t].at[pl.ds(s * SL, SL)])
        plsc.subcore_barrier()

        def group(g, cc):
            for rr in range(2):
                j = g * 2 + rr
                jw = jnp.where(j + 1 < nb, j + 1, 0)
                load(jw, 1 - rr)
                wait_load(j, rr)
                for t in range(T):
                    pltpu.sync_copy(tabs_sh[t].at[sidx[rr]], rows[t])
                for t in range(T):
                    pltpu.sync_copy(rows[t], accs[t].at[didx[rr]], add=True)
            return cc

        lax.fori_loop(0, nb // 2, group, 0)
        wait_load(0, 0)
        plsc.subcore_barrier()
        for t in range(T):
            pltpu.sync_copy(accs[t].at[pl.ds(s * SL, SL)],
                            outs[t].at[pl.ds(c * NPAD + s * SL, SL)])

    return agg_k


_LB = 2048


def _tc_norm(deg_p, x_t):
    G = NPAD // _LB

    def body(dref, xref, dinvref, uref):
        deg = dref[0:1, :] + dref[1:2, :] + 1.0
        dinv = lax.rsqrt(deg)
        dinvref[...] = dinv
        uref[...] = xref[...] * dinv

    return pl.pallas_call(
        body,
        grid=(G,),
        in_specs=[pl.BlockSpec((NC, _LB), lambda i: (0, i)),
                  pl.BlockSpec((2, _LB), lambda i: (0, i))],
        out_specs=[pl.BlockSpec((1, _LB), lambda i: (0, i)),
                   pl.BlockSpec((2, _LB), lambda i: (0, i))],
        out_shape=[jax.ShapeDtypeStruct((1, NPAD), jnp.float32),
                   jax.ShapeDtypeStruct((2, NPAD), jnp.float32)],
    )(deg_p, x_t)


def _tc_layer1(a0p, a1p, x_t, dinv, wpack):
    G = NPAD // _LB

    def body(a0r, a1r, xr, dr, wr, vr, gr):
        dinv = dr[...]
        d2 = dinv * dinv
        a0 = dinv * (a0r[0:1, :] + a0r[1:2, :]) + d2 * xr[0:1, :]
        a1 = dinv * (a1r[0:1, :] + a1r[1:2, :]) + d2 * xr[1:2, :]
        w = wr[...]
        h = jnp.maximum(w[:, 0:1] * a0 + w[:, 1:2] * a1 + w[:, 2:3], 0.0)
        g = jnp.sum(h * w[:, 3:4], axis=0, keepdims=True)
        gr[...] = g
        vr[...] = dinv * g

    return pl.pallas_call(
        body,
        grid=(G,),
        in_specs=[pl.BlockSpec((NC, _LB), lambda i: (0, i)),
                  pl.BlockSpec((NC, _LB), lambda i: (0, i)),
                  pl.BlockSpec((2, _LB), lambda i: (0, i)),
                  pl.BlockSpec((1, _LB), lambda i: (0, i)),
                  pl.BlockSpec((16, 4), lambda i: (0, 0))],
        out_specs=[pl.BlockSpec((1, _LB), lambda i: (0, i)),
                   pl.BlockSpec((1, _LB), lambda i: (0, i))],
        out_shape=[jax.ShapeDtypeStruct((1, NPAD), jnp.float32),
                   jax.ShapeDtypeStruct((1, NPAD), jnp.float32)],
    )(a0p, a1p, x_t, dinv, wpack)


def _tc_layer2(a2p, dinv, g, b2):
    G = NPAD // _LB

    def body(ar, dr, gr, br, outr):
        dinv = dr[...]
        outr[...] = dinv * (ar[0:1, :] + ar[1:2, :]) + dinv * dinv * gr[...] + br[...]

    return pl.pallas_call(
        body,
        grid=(G,),
        in_specs=[pl.BlockSpec((NC, _LB), lambda i: (0, i)),
                  pl.BlockSpec((1, _LB), lambda i: (0, i)),
                  pl.BlockSpec((1, _LB), lambda i: (0, i)),
                  pl.BlockSpec((1, 1), lambda i: (0, 0))],
        out_specs=pl.BlockSpec((1, _LB), lambda i: (0, i)),
        out_shape=jax.ShapeDtypeStruct((1, NPAD), jnp.float32),
    )(a2p, dinv, g, b2)


def kernel(x, edge_index, batch, W1, b1, W2, b2):
    N = x.shape[0]
    E = edge_index.shape[1]
    assert E % (NW * B) == 0 and N <= NPAD

    src1 = edge_index[0]
    dst1 = edge_index[1]
    zeros = jnp.zeros((NPAD,), jnp.float32)
    x_t = jnp.zeros((2, NPAD), jnp.float32).at[:, :N].set(x.T)

    deg_p = _make_deg_kernel(E)(dst1, zeros).reshape(NC, NPAD)
    dinv, u_t = _tc_norm(deg_p, x_t)

    a0p, a1p = _make_agg_kernel(E, 2)(
        src1, dst1, u_t[0].reshape(NPAD), u_t[1].reshape(NPAD), zeros)
    wpack = jnp.stack([W1[0], W1[1], b1, W2[:, 0]], axis=1)
    v, g = _tc_layer1(a0p.reshape(NC, NPAD), a1p.reshape(NC, NPAD),
                      x_t, dinv, wpack)

    (a2p,) = _make_agg_kernel(E, 1)(src1, dst1, v.reshape(NPAD), zeros)
    out = _tc_layer2(a2p.reshape(NC, NPAD), dinv, g, b2.reshape(1, 1))
    return out[0, :N].reshape(N, 1)


# speedup vs baseline: 1.5798x; 1.0283x over previous
import functools

import jax
import jax.numpy as jnp
from jax import lax
from jax.experimental import pallas as pl
from jax.experimental.pallas import tpu as pltpu
from jax.experimental.pallas import tpu_sc as plsc

NC, NS, L = 2, 16, 16
NW = NC * NS
B = 10000
NPAD = 102400

_MESH = plsc.VectorSubcoreMesh(
    core_axis_name="c", subcore_axis_name="s", num_cores=NC, num_subcores=NS)


def _make_deg_kernel(E):
    nb = E // NW // B
    SL = NPAD // NS
    assert nb % 4 == 0 and nb >= 4 and B % L == 0

    scratch = ([pltpu.VMEM((B,), jnp.int32) for _ in range(4)]
               + [pltpu.VMEM((B,), jnp.float32),
                  pltpu.VMEM_SHARED((NPAD,), jnp.float32)]
               + [pltpu.SemaphoreType.DMA for _ in range(5)])

    @functools.partial(
        pl.kernel,
        out_type=jax.ShapeDtypeStruct((NC * NPAD,), jnp.float32),
        mesh=_MESH,
        scratch_types=scratch,
    )
    def deg_k(dst1, zeros, out, *rest):
        didx = rest[0:4]
        ones_v, acc = rest[4], rest[5]
        ls = rest[6:10]
        ss = rest[10]
        c = lax.axis_index("c")
        s = lax.axis_index("s")
        base = (c * NS + s) * (nb * B)

        def load(j, r):
            pltpu.async_copy(dst1.at[pl.ds(base + j * B, B)], didx[r], ls[r])

        def wait_scatter(r):
            pltpu.make_async_copy(ones_v, acc.at[didx[r]], ss).wait()

        load(0, 0)
        pltpu.sync_copy(zeros.at[pl.ds(s * SL, SL)], acc.at[pl.ds(s * SL, SL)])
        for i in range(B // L):
            ones_v[pl.ds(i * L, L)] = jnp.ones((L,), jnp.float32)
        plsc.subcore_barrier()

        def group(g, cc):
            for rr in range(4):
                j = g * 4 + rr
                jw = jnp.where(j + 1 < nb, j + 1, 0)
                load(jw, (rr + 1) % 4)
                pltpu.make_async_copy(
                    dst1.at[pl.ds(base + j * B, B)], didx[rr], ls[rr]).wait()
                if rr == 0:
                    @pl.when(g > 0)
                    def _():
                        wait_scatter(3)
                else:
                    wait_scatter(rr - 1)
                pltpu.async_copy(ones_v, acc.at[didx[rr]], ss, add=True)
            return cc

        lax.fori_loop(0, nb // 4, group, 0)
        wait_scatter(3)
        pltpu.make_async_copy(dst1.at[pl.ds(base, B)], didx[0], ls[0]).wait()
        plsc.subcore_barrier()
        pltpu.sync_copy(acc.at[pl.ds(s * SL, SL)],
                        out.at[pl.ds(c * NPAD + s * SL, SL)])

    return deg_k


def _make_agg_kernel(E, T):
    nb = E // NW // B
    SL = NPAD // NS
    assert nb % 4 == 0 and nb >= 4 and B % L == 0

    scratch = ([pltpu.VMEM((B,), jnp.int32) for _ in range(2)]
               + [pltpu.VMEM((B,), jnp.int32) for _ in range(4)]
               + [pltpu.VMEM((B,), jnp.float32) for _ in range(2 * T)]
               + [pltpu.VMEM_SHARED((NPAD,), jnp.float32) for _ in range(2 * T)]
               + [pltpu.SemaphoreType.DMA for _ in range(5)])

    @functools.partial(
        pl.kernel,
        out_type=tuple(jax.ShapeDtypeStruct((NC * NPAD,), jnp.float32)
                       for _ in range(T)),
        mesh=_MESH,
        scratch_types=scratch,
    )
    def agg_k(src1, dst1, *rest):
        tabs = rest[:T]
        zeros = rest[T]
        outs = rest[T + 1:T + 1 + T]
        sc = list(rest[T + 1 + T:])
        sidx = sc[0:2]
        didx = sc[2:6]
        rows = [sc[6 + 2 * t:8 + 2 * t] for t in range(T)]
        accs = sc[6 + 2 * T:6 + 3 * T]
        tabs_sh = sc[6 + 3 * T:6 + 4 * T]
        ls = sc[6 + 4 * T:10 + 4 * T]
        ss = sc[10 + 4 * T]
        c = lax.axis_index("c")
        s = lax.axis_index("s")
        base = (c * NS + s) * (nb * B)

        def load(j, r2, r4):
            pltpu.async_copy(src1.at[pl.ds(base + j * B, B)], sidx[r2], ls[r4])
            pltpu.async_copy(dst1.at[pl.ds(base + j * B, B)], didx[r4], ls[r4])

        def wait_load(j, r2, r4):
            pltpu.make_async_copy(
                src1.at[pl.ds(base + j * B, B)], sidx[r2], ls[r4]).wait()
            pltpu.make_async_copy(
                dst1.at[pl.ds(base + j * B, B)], didx[r4], ls[r4]).wait()

        def wait_scatter(r2, r4):
            for t in range(T):
                pltpu.make_async_copy(
                    rows[t][r2], accs[t].at[didx[r4]], ss).wait()

        load(0, 0, 0)
        for t in range(T):
            pltpu.sync_copy(zeros.at[pl.ds(s * SL, SL)],
                            accs[t].at[pl.ds(s * SL, SL)])
            pltpu.sync_copy(tabs[t].at[pl.ds(s * SL, SL)],
                            tabs_sh[t].at[pl.ds(s * SL, SL)])
        plsc.subcore_barrier()

        def group(g, cc):
            for rr in range(4):
                j = g * 4 + rr
                r2 = rr % 2
                jw = jnp.where(j + 1 < nb, j + 1, 0)
                load(jw, 1 - r2, (rr + 1) % 4)
                wait_load(j, r2, rr)
                for t in range(T):
                    pltpu.sync_copy(tabs_sh[t].at[sidx[r2]], rows[t][r2])
                if rr == 0:
                    @pl.when(g > 0)
                    def _():
                        wait_scatter(1, 3)
                else:
                    wait_scatter(1 - r2, rr - 1)
                for t in range(T):
                    pltpu.async_copy(rows[t][r2], accs[t].at[didx[rr]],
                                     ss, add=True)
            return cc

        lax.fori_loop(0, nb // 4, group, 0)
        wait_scatter(1, 3)
        wait_load(0, 0, 0)
        plsc.subcore_barrier()
        for t in range(T):
            pltpu.sync_copy(accs[t].at[pl.ds(s * SL, SL)],
                            outs[t].at[pl.ds(c * NPAD + s * SL, SL)])

    return agg_k


_LB = 2048


def _tc_norm(deg_p, x_t):
    G = NPAD // _LB

    def body(dref, xref, dinvref, uref):
        deg = dref[0:1, :] + dref[1:2, :] + 1.0
        dinv = lax.rsqrt(deg)
        dinvref[...] = dinv
        uref[...] = xref[...] * dinv

    return pl.pallas_call(
        body,
        grid=(G,),
        in_specs=[pl.BlockSpec((NC, _LB), lambda i: (0, i)),
                  pl.BlockSpec((2, _LB), lambda i: (0, i))],
        out_specs=[pl.BlockSpec((1, _LB), lambda i: (0, i)),
                   pl.BlockSpec((2, _LB), lambda i: (0, i))],
        out_shape=[jax.ShapeDtypeStruct((1, NPAD), jnp.float32),
                   jax.ShapeDtypeStruct((2, NPAD), jnp.float32)],
    )(deg_p, x_t)


def _tc_layer1(a0p, a1p, x_t, dinv, wpack):
    G = NPAD // _LB

    def body(a0r, a1r, xr, dr, wr, vr, gr):
        dinv = dr[...]
        d2 = dinv * dinv
        a0 = dinv * (a0r[0:1, :] + a0r[1:2, :]) + d2 * xr[0:1, :]
        a1 = dinv * (a1r[0:1, :] + a1r[1:2, :]) + d2 * xr[1:2, :]
        w = wr[...]
        h = jnp.maximum(w[:, 0:1] * a0 + w[:, 1:2] * a1 + w[:, 2:3], 0.0)
        g = jnp.sum(h * w[:, 3:4], axis=0, keepdims=True)
        gr[...] = g
        vr[...] = dinv * g

    return pl.pallas_call(
        body,
        grid=(G,),
        in_specs=[pl.BlockSpec((NC, _LB), lambda i: (0, i)),
                  pl.BlockSpec((NC, _LB), lambda i: (0, i)),
                  pl.BlockSpec((2, _LB), lambda i: (0, i)),
                  pl.BlockSpec((1, _LB), lambda i: (0, i)),
                  pl.BlockSpec((16, 4), lambda i: (0, 0))],
        out_specs=[pl.BlockSpec((1, _LB), lambda i: (0, i)),
                   pl.BlockSpec((1, _LB), lambda i: (0, i))],
        out_shape=[jax.ShapeDtypeStruct((1, NPAD), jnp.float32),
                   jax.ShapeDtypeStruct((1, NPAD), jnp.float32)],
    )(a0p, a1p, x_t, dinv, wpack)


def _tc_layer2(a2p, dinv, g, b2):
    G = NPAD // _LB

    def body(ar, dr, gr, br, outr):
        dinv = dr[...]
        outr[...] = dinv * (ar[0:1, :] + ar[1:2, :]) + dinv * dinv * gr[...] + br[...]

    return pl.pallas_call(
        body,
        grid=(G,),
        in_specs=[pl.BlockSpec((NC, _LB), lambda i: (0, i)),
                  pl.BlockSpec((1, _LB), lambda i: (0, i)),
                  pl.BlockSpec((1, _LB), lambda i: (0, i)),
                  pl.BlockSpec((1, 1), lambda i: (0, 0))],
        out_specs=pl.BlockSpec((1, _LB), lambda i: (0, i)),
        out_shape=jax.ShapeDtypeStruct((1, NPAD), jnp.float32),
    )(a2p, dinv, g, b2)


def kernel(x, edge_index, batch, W1, b1, W2, b2):
    N = x.shape[0]
    E = edge_index.shape[1]
    assert E % (NW * B) == 0 and N <= NPAD

    src1 = edge_index[0]
    dst1 = edge_index[1]
    zeros = jnp.zeros((NPAD,), jnp.float32)
    x_t = jnp.zeros((2, NPAD), jnp.float32).at[:, :N].set(x.T)

    deg_p = _make_deg_kernel(E)(dst1, zeros).reshape(NC, NPAD)
    dinv, u_t = _tc_norm(deg_p, x_t)

    a0p, a1p = _make_agg_kernel(E, 2)(
        src1, dst1, u_t[0].reshape(NPAD), u_t[1].reshape(NPAD), zeros)
    wpack = jnp.stack([W1[0], W1[1], b1, W2[:, 0]], axis=1)
    v, g = _tc_layer1(a0p.reshape(NC, NPAD), a1p.reshape(NC, NPAD),
                      x_t, dinv, wpack)

    (a2p,) = _make_agg_kernel(E, 1)(src1, dst1, v.reshape(NPAD), zeros)
    out = _tc_layer2(a2p.reshape(NC, NPAD), dinv, g, b2.reshape(1, 1))
    return out[0, :N].reshape(N, 1)
